# Initial kernel scaffold; baseline (speedup 1.0000x reference)
import jax
import jax.numpy as jnp
from jax.experimental import pallas as pl


def kernel(input):
    raise NotImplementedError("write your pallas kernel here")


# speedup vs baseline: 6.5258x; 6.5258x over previous
import functools

import jax
import jax.numpy as jnp
from jax import lax
from jax.experimental import pallas as pl
from jax.experimental.pallas import tpu as pltpu
from jax.experimental.pallas import tpu_sc as plsc

_L = 16
_ROWS = 64
_N = 8192
_CHUNKS = _N // _L
_WORKERS = 32
_ROWS_PER_W = _ROWS // _WORKERS
_BISECT_ITERS = 32
_PAD = -1e30


def _row_sparsemax(xbuf, r, cand):
    pad16 = jnp.full((_L,), _PAD, jnp.float32)

    def p1(i, acc):
        v = xbuf[r, pl.ds(i * _L, _L)]
        cand[pl.ds(i * _L, _L)] = pad16
        return jnp.maximum(acc, v)

    m16 = lax.fori_loop(0, _CHUNKS, p1, pad16)
    cand[pl.ds(_N, _L)] = pad16
    m = jnp.max(m16)
    thr = m - 1.0

    def p2(i, off):
        v = xbuf[r, pl.ds(i * _L, _L)]
        msk = v > thr
        ones = jnp.where(msk, 1.0, 0.0).astype(jnp.float32)
        c = jnp.cumsum(ones)
        idx = c.astype(jnp.int32) + (off - 1)
        plsc.store_scatter(cand, [idx], v, mask=msk)
        return off + jnp.max(c).astype(jnp.int32)

    cnt = lax.fori_loop(0, _CHUNKS, p2, jnp.int32(0))
    ncv = (cnt + (_L - 1)) >> 4

    def bis(_, lohi):
        lo, hi = lohi
        mid = 0.5 * (lo + hi)

        def acc_body(i, acc):
            v = cand[pl.ds(i * _L, _L)]
            return acc + jnp.maximum(v - mid, 0.0)

        f = jnp.sum(lax.fori_loop(0, ncv, acc_body,
                                  jnp.zeros((_L,), jnp.float32)))
        gt = f > 1.0
        return jnp.where(gt, mid, lo), jnp.where(gt, hi, mid)

    lo, hi = lax.fori_loop(0, _BISECT_ITERS, bis, (thr, m))
    tg = 0.5 * (lo + hi)

    def p3(i, carry):
        sv, kv = carry
        v = cand[pl.ds(i * _L, _L)]
        msk = v > tg
        return (sv + jnp.where(msk, v, 0.0),
                kv + jnp.where(msk, 1.0, 0.0))

    sv, kv = lax.fori_loop(
        0, ncv, p3,
        (jnp.zeros((_L,), jnp.float32), jnp.zeros((_L,), jnp.float32)))
    num = jnp.broadcast_to(jnp.sum(sv) - 1.0, (_L,))
    den = jnp.broadcast_to(jnp.maximum(jnp.sum(kv), 1.0), (_L,))
    tau16 = num / den

    def p4(i, _):
        s = pl.ds(i * _L, _L)
        xbuf[r, s] = jnp.maximum(xbuf[r, s] - tau16, 0.0)
        return 0

    lax.fori_loop(0, _CHUNKS, p4, 0)


def kernel(input):
    x = input
    mesh = plsc.VectorSubcoreMesh(core_axis_name="c", subcore_axis_name="s")

    @functools.partial(
        pl.kernel,
        mesh=mesh,
        out_type=jax.ShapeDtypeStruct((_ROWS, _N), jnp.float32),
        scratch_types=[
            pltpu.VMEM((_ROWS_PER_W, _N), jnp.float32),
            pltpu.VMEM((_N + _L,), jnp.float32),
        ],
        compiler_params=pltpu.CompilerParams(needs_layout_passes=False),
    )
    def run(x_hbm, out_hbm, xbuf, cand):
        wid = lax.axis_index("s") * 2 + lax.axis_index("c")
        base = wid * _ROWS_PER_W
        pltpu.sync_copy(x_hbm.at[pl.ds(base, _ROWS_PER_W)], xbuf)
        for r in range(_ROWS_PER_W):
            _row_sparsemax(xbuf, r, cand)
        pltpu.sync_copy(xbuf, out_hbm.at[pl.ds(base, _ROWS_PER_W)])

    return run(x)


# speedup vs baseline: 7.7618x; 1.1894x over previous
import functools

import jax
import jax.numpy as jnp
from jax import lax
from jax.experimental import pallas as pl
from jax.experimental.pallas import tpu as pltpu
from jax.experimental.pallas import tpu_sc as plsc

_L = 16
_ROWS = 64
_N = 8192
_CHUNKS = _N // _L
_WORKERS = 32
_ROWS_PER_W = _ROWS // _WORKERS
_BISECT_ITERS = 32
_UNROLL = 8
_PAD = -1e30


def _row_sparsemax(xbuf, r, cand):
    pad16 = jnp.full((_L,), _PAD, jnp.float32)

    def p1(i, acc):
        base = i * (_L * _UNROLL)
        vs = [xbuf[r, pl.ds(base + u * _L, _L)] for u in range(_UNROLL)]
        while len(vs) > 1:
            vs = [jnp.maximum(vs[2 * j], vs[2 * j + 1])
                  for j in range(len(vs) // 2)]
        return jnp.maximum(acc, vs[0])

    m16 = lax.fori_loop(0, _CHUNKS // _UNROLL, p1, pad16)
    m = jnp.max(m16)
    thr = m - 1.0

    def p2(i, off16):
        base = i * (_L * 4)
        for u in range(4):
            v = xbuf[r, pl.ds(base + u * _L, _L)]
            msk = v > thr
            ones = jnp.where(msk, 1.0, 0.0).astype(jnp.float32)
            c = jnp.cumsum(ones)
            idx = (c.astype(jnp.int32) - 1) + off16
            plsc.store_scatter(cand, [idx], v, mask=msk)
            off16 = off16 + plsc.all_reduce_population_count(msk)
        return off16

    off16 = lax.fori_loop(0, _CHUNKS // 4, p2, jnp.zeros((_L,), jnp.int32))
    cnt = jnp.max(off16.astype(jnp.float32)).astype(jnp.int32)
    cand[pl.ds(cnt, _L)] = pad16
    ncv = (cnt + (_L - 1)) >> 4

    def bis(_, lohi):
        lo, hi = lohi
        mid = 0.5 * (lo + hi)

        def acc_body(i, acc):
            v = cand[pl.ds(i * _L, _L)]
            return acc + jnp.maximum(v - mid, 0.0)

        f = jnp.sum(lax.fori_loop(0, ncv, acc_body,
                                  jnp.zeros((_L,), jnp.float32)))
        gt = f > 1.0
        return jnp.where(gt, mid, lo), jnp.where(gt, hi, mid)

    lo, hi = lax.fori_loop(0, _BISECT_ITERS, bis, (thr, m))
    tg = 0.5 * (lo + hi)

    def p3(i, carry):
        sv, kv = carry
        v = cand[pl.ds(i * _L, _L)]
        msk = v > tg
        return (sv + jnp.where(msk, v, 0.0),
                kv + jnp.where(msk, 1.0, 0.0))

    sv, kv = lax.fori_loop(
        0, ncv, p3,
        (jnp.zeros((_L,), jnp.float32), jnp.zeros((_L,), jnp.float32)))
    num = jnp.broadcast_to(jnp.sum(sv) - 1.0, (_L,))
    den = jnp.broadcast_to(jnp.maximum(jnp.sum(kv), 1.0), (_L,))
    tau16 = num / den

    def p4(i, _):
        base = i * (_L * _UNROLL)
        for u in range(_UNROLL):
            s = pl.ds(base + u * _L, _L)
            xbuf[r, s] = jnp.maximum(xbuf[r, s] - tau16, 0.0)
        return 0

    lax.fori_loop(0, _CHUNKS // _UNROLL, p4, 0)


def kernel(input):
    x = input
    mesh = plsc.VectorSubcoreMesh(core_axis_name="c", subcore_axis_name="s")

    @functools.partial(
        pl.kernel,
        mesh=mesh,
        out_type=jax.ShapeDtypeStruct((_ROWS, _N), jnp.float32),
        scratch_types=[
            pltpu.VMEM((_ROWS_PER_W, _N), jnp.float32),
            pltpu.VMEM((_N + _L,), jnp.float32),
        ],
        compiler_params=pltpu.CompilerParams(needs_layout_passes=False),
    )
    def run(x_hbm, out_hbm, xbuf, cand):
        wid = lax.axis_index("s") * 2 + lax.axis_index("c")
        base = wid * _ROWS_PER_W
        pltpu.sync_copy(x_hbm.at[pl.ds(base, _ROWS_PER_W)], xbuf)
        for r in range(_ROWS_PER_W):
            _row_sparsemax(xbuf, r, cand)
        pltpu.sync_copy(xbuf, out_hbm.at[pl.ds(base, _ROWS_PER_W)])

    return run(x)


# speedup vs baseline: 8.7291x; 1.1246x over previous
import functools

import jax
import jax.numpy as jnp
from jax import lax
from jax.experimental import pallas as pl
from jax.experimental.pallas import tpu as pltpu
from jax.experimental.pallas import tpu_sc as plsc

_L = 16
_ROWS = 64
_N = 8192
_CHUNKS = _N // _L
_WORKERS = 32
_ROWS_PER_W = _ROWS // _WORKERS
_BISECT_ITERS = 30
_G = 4
_NGROUPS = _CHUNKS // _G
_PAD = -1e30


def _row_sparsemax(xbuf, r, cand, candix, gmax, obuf):
    pad16 = jnp.full((_L,), _PAD, jnp.float32)
    lane = lax.iota(jnp.int32, _L)

    def p1(i, acc):
        base = i * (_L * 8)
        vs = [xbuf[r, pl.ds(base + u * _L, _L)] for u in range(8)]
        l1 = [jnp.maximum(vs[2 * j], vs[2 * j + 1]) for j in range(4)]
        l2 = [jnp.maximum(l1[0], l1[1]), jnp.maximum(l1[2], l1[3])]
        gmax[pl.ds(i * (2 * _L), _L)] = l2[0]
        gmax[pl.ds(i * (2 * _L) + _L, _L)] = l2[1]
        return jnp.maximum(acc, jnp.maximum(l2[0], l2[1]))

    m16 = lax.fori_loop(0, _CHUNKS // 8, p1, pad16)
    m = jnp.max(m16)
    thr = m - 1.0

    def p2(g, off16):
        mv = gmax[pl.ds(g * _L, _L)]

        def do(off):
            for u in range(_G):
                el = g * (_G * _L) + u * _L
                v = xbuf[r, pl.ds(el, _L)]
                msk = v > thr
                ones = jnp.where(msk, 1.0, 0.0).astype(jnp.float32)
                c = plsc.cumsum(ones)
                pos = (c.astype(jnp.int32) - 1) + off
                plsc.store_scatter(cand, [pos], v, mask=msk)
                plsc.store_scatter(candix, [pos], lane + el, mask=msk)
                off = off + plsc.all_reduce_population_count(msk)
            return off

        return lax.cond(jnp.any(mv > thr), do, lambda o: o, off16)

    off16 = lax.fori_loop(0, _NGROUPS, p2, jnp.zeros((_L,), jnp.int32))
    cnt = jnp.max(off16.astype(jnp.float32)).astype(jnp.int32)
    cand[pl.ds(cnt, _L)] = pad16
    ncv = (cnt + (_L - 1)) >> 4

    def bis(_, lohi):
        lo, hi = lohi
        mid = 0.5 * (lo + hi)

        def acc_body(i, acc):
            v = cand[pl.ds(i * _L, _L)]
            return acc + jnp.maximum(v - mid, 0.0)

        f = jnp.sum(lax.fori_loop(0, ncv, acc_body,
                                  jnp.zeros((_L,), jnp.float32)))
        gt = f > 1.0
        return jnp.where(gt, mid, lo), jnp.where(gt, hi, mid)

    lo, hi = lax.fori_loop(0, _BISECT_ITERS, bis, (thr, m))
    tg = 0.5 * (lo + hi)

    def p3(i, carry):
        sv, kv = carry
        v = cand[pl.ds(i * _L, _L)]
        msk = v > tg
        return (sv + jnp.where(msk, v, 0.0),
                kv + jnp.where(msk, 1.0, 0.0))

    sv, kv = lax.fori_loop(
        0, ncv, p3,
        (jnp.zeros((_L,), jnp.float32), jnp.zeros((_L,), jnp.float32)))
    num = jnp.broadcast_to(jnp.sum(sv) - 1.0, (_L,))
    den = jnp.broadcast_to(jnp.maximum(jnp.sum(kv), 1.0), (_L,))
    tau16 = num / den

    row16 = jnp.full((_L,), r, jnp.int32)

    def pout(i, _):
        v = cand[pl.ds(i * _L, _L)]
        ix = candix[pl.ds(i * _L, _L)]
        msk = v > tg
        plsc.store_scatter(obuf, [row16, ix],
                           jnp.maximum(v - tau16, 0.0), mask=msk)
        return 0

    lax.fori_loop(0, ncv, pout, 0)


def kernel(input):
    x = input
    mesh = plsc.VectorSubcoreMesh(core_axis_name="c", subcore_axis_name="s")

    @functools.partial(
        pl.kernel,
        mesh=mesh,
        out_type=jax.ShapeDtypeStruct((_ROWS, _N), jnp.float32),
        scratch_types=[
            pltpu.VMEM((_ROWS_PER_W, _N), jnp.float32),
            pltpu.VMEM((_ROWS_PER_W, _N), jnp.float32),
            pltpu.VMEM((_N + _L,), jnp.float32),
            pltpu.VMEM((_N + _L,), jnp.int32),
            pltpu.VMEM((_NGROUPS * _L,), jnp.float32),
            pltpu.SemaphoreType.DMA,
            pltpu.SemaphoreType.DMA,
        ],
        compiler_params=pltpu.CompilerParams(needs_layout_passes=False),
    )
    def run(x_hbm, out_hbm, xbuf, obuf, cand, candix, gmax, sem_in, sem_out):
        wid = lax.axis_index("s") * 2 + lax.axis_index("c")
        base = wid * _ROWS_PER_W
        cp_in = pltpu.async_copy(x_hbm.at[pl.ds(base, _ROWS_PER_W)],
                                 xbuf, sem_in)

        zero16 = jnp.zeros((_L,), jnp.float32)

        def z(i, _):
            b = i * (_L * 8)
            for rr in range(_ROWS_PER_W):
                for u in range(8):
                    obuf[rr, pl.ds(b + u * _L, _L)] = zero16
            return 0

        lax.fori_loop(0, _CHUNKS // 8, z, 0)
        cp_in.wait()

        cps = []
        for r in range(_ROWS_PER_W):
            _row_sparsemax(xbuf, r, cand, candix, gmax, obuf)
            cp = pltpu.async_copy(obuf.at[pl.ds(r, 1)],
                                  out_hbm.at[pl.ds(base + r, 1)], sem_out)
            cps.append(cp)
        for cp in cps:
            cp.wait()

    return run(x)
